# baseline (speedup 1.0000x reference)
import jax
import jax.numpy as jnp
import numpy as np
from jax import lax
from jax.experimental import pallas as pl
from jax.experimental.pallas import tpu as pltpu
from jax.experimental.pallas import tpu_sc as plsc

B = 4
N = 16800
L = 16
NC = 2
NS = 16
WPB = 8
C = N // WPB
U = 4
NV = -(-C // L)
VAR0 = np.float32(0.1)
NEG_INF = np.float32(-np.inf)
IMAX = np.int32(2**31 - 1)

_MESH = plsc.VectorSubcoreMesh(
    core_axis_name="c", subcore_axis_name="s", num_cores=NC, num_subcores=NS
)


def _sc_body(conf_hbm, landms_hbm, priors_hbm, out_hbm,
             sbuf, mstage, istage, mload, iload, tmpf, tmpi, lrow, prow, obuf,
             sem_l, sem_p):
    c = lax.axis_index("c")
    s = lax.axis_index("s")
    g = s // WPB
    w = s % WPB
    b = 2 * c + g
    base = w * C

    pltpu.sync_copy(conf_hbm.at[b, pl.ds(base * 2, C * 2)], sbuf.at[pl.ds(0, C * 2)])

    lane = lax.iota(jnp.int32, L)

    def scan_vec(j, carry):
        run_max, run_idx = carry
        n = j * L + lane
        v = plsc.load_gather(sbuf, [n * 2 + 1])
        v = jnp.where(v > 0.0, v, NEG_INF)
        upd = v > run_max
        return jnp.where(upd, v, run_max), jnp.where(upd, base + n, run_idx)

    def step(i, chains):
        return tuple(scan_vec(i * U + k, chains[k]) for k in range(U))

    init = tuple(
        (jnp.full((L,), NEG_INF, jnp.float32), jnp.zeros((L,), jnp.int32))
        for _ in range(U)
    )
    nfull = (NV - 1) // U
    chains = lax.fori_loop(0, nfull, step, init)

    chains = tuple(
        scan_vec(nfull * U + k, chains[k]) if nfull * U + k < NV - 1 else chains[k]
        for k in range(U)
    )

    run_max, run_idx = chains[0]
    for m2, i2 in chains[1:]:
        upd = (m2 > run_max) | ((m2 == run_max) & (i2 < run_idx))
        run_max = jnp.where(upd, m2, run_max)
        run_idx = jnp.where(upd, i2, run_idx)

    n = (NV - 1) * L + lane
    v = plsc.load_gather(sbuf, [jnp.minimum(n, C - 1) * 2 + 1])
    v = jnp.where((v > 0.0) & (n < C), v, NEG_INF)
    upd = (v > run_max) | ((v == run_max) & (base + n < run_idx))
    run_max = jnp.where(upd, v, run_max)
    run_idx = jnp.where(upd, base + n, run_idx)

    tmpf[...] = run_max
    tmpi[...] = run_idx
    pltpu.sync_copy(tmpf, mstage.at[pl.ds(s * L, L)])
    pltpu.sync_copy(tmpi, istage.at[pl.ds(s * L, L)])
    plsc.subcore_barrier()

    @pl.when(w == 0)
    def _():
        pltpu.sync_copy(mstage.at[pl.ds(g * WPB * L, WPB * L)], mload)
        pltpu.sync_copy(istage.at[pl.ds(g * WPB * L, WPB * L)], iload)
        best_m = mload[pl.ds(0, L)]
        best_i = iload[pl.ds(0, L)]
        for k in range(1, WPB):
            m2 = mload[pl.ds(k * L, L)]
            i2 = iload[pl.ds(k * L, L)]
            upd = (m2 > best_m) | ((m2 == best_m) & (i2 < best_i))
            best_m = jnp.where(upd, m2, best_m)
            best_i = jnp.where(upd, i2, best_i)
        top = jnp.max(best_m, axis=0)
        cand = jnp.where(best_m == top, best_i, IMAX)
        idx = jnp.min(cand, axis=0)

        cl = pltpu.async_copy(landms_hbm.at[b, pl.ds(idx, 1), :], lrow, sem_l)
        cp = pltpu.async_copy(priors_hbm.at[pl.ds(idx, 1), :], prow, sem_p)
        cl.wait()
        cp.wait()

        zeros = jnp.zeros((L,), jnp.int32)
        par = lane & 1
        lvec = plsc.load_gather(lrow, [zeros, jnp.minimum(lane, 9)])
        pxy = plsc.load_gather(prow, [zeros, par])
        pwh = plsc.load_gather(prow, [zeros, par + 2])

        obuf[...] = pxy + lvec * VAR0 * pwh
        pltpu.sync_copy(obuf, out_hbm.at[b])


_sc_call = pl.kernel(
    _sc_body,
    out_type=jax.ShapeDtypeStruct((B, L), jnp.float32),
    mesh=_MESH,
    compiler_params=pltpu.CompilerParams(
        needs_layout_passes=False, use_tc_tiling_on_sc=False
    ),
    scratch_types=[
        pltpu.VMEM((NV * L * 2,), jnp.float32),
        pltpu.VMEM_SHARED((NS * L,), jnp.float32),
        pltpu.VMEM_SHARED((NS * L,), jnp.int32),
        pltpu.VMEM((WPB * L,), jnp.float32),
        pltpu.VMEM((WPB * L,), jnp.int32),
        pltpu.VMEM((L,), jnp.float32),
        pltpu.VMEM((L,), jnp.int32),
        pltpu.VMEM((1, 10), jnp.float32),
        pltpu.VMEM((1, 4), jnp.float32),
        pltpu.VMEM((L,), jnp.float32),
        pltpu.SemaphoreType.DMA,
        pltpu.SemaphoreType.DMA,
    ],
)


def kernel(loc, conf, landms, priors):
    del loc
    out = _sc_call(conf.reshape(B, 2 * N), landms, priors)
    return out[:, :10]
